# baseline (speedup 1.0000x reference)
import functools

import jax
import jax.numpy as jnp
from jax import lax
from jax.experimental import pallas as pl
from jax.experimental.pallas import tpu as pltpu
from jax.experimental.pallas import tpu_sc as plsc

N = 10000
EDGES = 320000
D = 128
NC, NS = 2, 16
NW = NC * NS
EPT = EDGES // NW
CHUNK = 64
NCHUNK = 160
EPT_P = NCHUNK * CHUNK
EDGES_P = NW * EPT_P
RING = 4
PDIST = 2
NP = N + 8
ROWS_PT = 624
TAIL_ROWS = N - NS * ROWS_PT

_MESH = plsc.VectorSubcoreMesh(
    core_axis_name="c", subcore_axis_name="s", num_cores=NC, num_subcores=NS)


@functools.partial(
    pl.kernel,
    out_type=jax.ShapeDtypeStruct((NW, N), jnp.float32),
    mesh=_MESH,
    scratch_types=[
        pltpu.VMEM((EPT,), jnp.int32),
        pltpu.VMEM((N,), jnp.float32),
    ],
    compiler_params=pltpu.CompilerParams(needs_layout_passes=False),
)
def _deg_kernel(dst_hbm, out_hbm, dbuf, acc):
    cid = lax.axis_index("c")
    sid = lax.axis_index("s")
    wid = cid * NS + sid
    pltpu.sync_copy(dst_hbm.at[wid], dbuf)
    zeros16 = jnp.zeros((16,), jnp.float32)

    @pl.loop(0, N // 16)
    def _zero(i):
        acc[pl.ds(i * 16, 16)] = zeros16

    ones16 = jnp.ones((16,), jnp.float32)

    @pl.loop(0, EPT // 16)
    def _hist(g):
        idx = dbuf[pl.ds(g * 16, 16)]
        plsc.addupdate_scatter(acc, [idx], ones16)

    pltpu.sync_copy(acc, out_hbm.at[wid])


@functools.partial(
    pl.kernel,
    out_type=jax.ShapeDtypeStruct((NC, N, D), jnp.float32),
    mesh=_MESH,
    scratch_types=[
        pltpu.VMEM_SHARED((N, D), jnp.float32),
        pltpu.VMEM((EPT_P,), jnp.int32),
        pltpu.VMEM((RING, CHUNK), jnp.int32),
        pltpu.VMEM((RING, CHUNK, D), jnp.float32),
        pltpu.SemaphoreType.DMA((RING,)),
        pltpu.SemaphoreType.DMA((RING,)),
        pltpu.SemaphoreType.DMA((RING,)),
    ],
)
def _hop_kernel(y_hbm, zero_hbm, src_hbm, dst_hbm, out_hbm,
                acc, sidx, didx, rows, gsem, isem, ssem):
    cid = lax.axis_index("c")
    sid = lax.axis_index("s")
    wid = cid * NS + sid
    row0 = pl.multiple_of(sid * ROWS_PT, 8)
    ebase = pl.multiple_of(wid * EPT_P, 64)

    def start_chunk(jj, b):
        eoff = pl.multiple_of(ebase + jj * CHUNK, 64)
        pltpu.async_copy(dst_hbm.at[pl.ds(eoff, CHUNK)], didx.at[b],
                         isem.at[b])
        loff = pl.multiple_of(jj * CHUNK, 64)
        pltpu.async_copy(y_hbm.at[sidx.at[pl.ds(loff, CHUNK)]], rows.at[b],
                         gsem.at[b])

    def wait_chunk(jj, b):
        eoff = pl.multiple_of(ebase + jj * CHUNK, 64)
        pltpu.make_async_copy(dst_hbm.at[pl.ds(eoff, CHUNK)], didx.at[b],
                              isem.at[b]).wait()
        loff = pl.multiple_of(jj * CHUNK, 64)
        pltpu.make_async_copy(y_hbm.at[sidx.at[pl.ds(loff, CHUNK)]],
                              rows.at[b], gsem.at[b]).wait()

    pltpu.sync_copy(src_hbm.at[pl.ds(ebase, EPT_P)], sidx)
    for b in range(PDIST):
        start_chunk(b, b)

    @pl.when(cid == 0)
    def _():
        pltpu.sync_copy(y_hbm.at[pl.ds(row0, ROWS_PT)],
                        acc.at[pl.ds(row0, ROWS_PT)])

        @pl.when(sid == NS - 1)
        def _():
            pltpu.sync_copy(y_hbm.at[pl.ds(NS * ROWS_PT, TAIL_ROWS)],
                            acc.at[pl.ds(NS * ROWS_PT, TAIL_ROWS)])

    @pl.when(cid != 0)
    def _():
        pltpu.sync_copy(zero_hbm.at[pl.ds(row0, ROWS_PT)],
                        acc.at[pl.ds(row0, ROWS_PT)])

        @pl.when(sid == NS - 1)
        def _():
            pltpu.sync_copy(zero_hbm.at[pl.ds(NS * ROWS_PT, TAIL_ROWS)],
                            acc.at[pl.ds(NS * ROWS_PT, TAIL_ROWS)])

    plsc.subcore_barrier()

    @pl.loop(0, NCHUNK, step=RING)
    def _main(j):
        for b in range(RING):
            k = j + b
            q = (b + PDIST) % RING
            kp = k + PDIST

            @pl.when(kp < NCHUNK)
            def _():
                @pl.when(k >= RING - PDIST)
                def _():
                    pltpu.make_async_copy(
                        rows.at[q], acc.at[didx.at[q]], ssem.at[q]).wait()

                start_chunk(kp, q)

            wait_chunk(k, b)
            pltpu.async_copy(rows.at[b], acc.at[didx.at[b]], ssem.at[b],
                             add=True)

    for b in range(RING):
        pltpu.make_async_copy(
            rows.at[b], acc.at[didx.at[b]], ssem.at[b]).wait()

    plsc.subcore_barrier()
    pltpu.sync_copy(acc.at[pl.ds(row0, ROWS_PT)],
                    out_hbm.at[cid, pl.ds(row0, ROWS_PT)])

    @pl.when(sid == NS - 1)
    def _():
        pltpu.sync_copy(acc.at[pl.ds(NS * ROWS_PT, TAIL_ROWS)],
                        out_hbm.at[cid, pl.ds(NS * ROWS_PT, TAIL_ROWS)])


def _deg_from(dp):
    return jnp.sum(dp, axis=0) + 1.0


def _prep_body(x_ref, w_ref, dp_ref, y0_ref):
    dinv = lax.rsqrt(_deg_from(dp_ref[...]))
    xw = lax.dot_general(x_ref[...], w_ref[...], (((1,), (1,)), ((), ())),
                         preferred_element_type=jnp.float32)
    y0_ref[pl.ds(0, N)] = dinv[:, None] * xw
    y0_ref[pl.ds(N, NP - N)] = jnp.zeros((NP - N, D), jnp.float32)


def _comb1_body(p_ref, dp_ref, y1_ref):
    deg = _deg_from(dp_ref[...])
    y1_ref[pl.ds(0, N)] = (p_ref[0] + p_ref[1]) / deg[:, None]
    y1_ref[pl.ds(N, NP - N)] = jnp.zeros((NP - N, D), jnp.float32)


def _comb2_body(p_ref, dp_ref, b_ref, o_ref):
    dinv = lax.rsqrt(_deg_from(dp_ref[...]))
    o_ref[...] = dinv[:, None] * (p_ref[0] + p_ref[1]) + b_ref[...]


_prep = pl.pallas_call(
    _prep_body, out_shape=jax.ShapeDtypeStruct((NP, D), jnp.float32))
_comb1 = pl.pallas_call(
    _comb1_body, out_shape=jax.ShapeDtypeStruct((NP, D), jnp.float32))
_comb2 = pl.pallas_call(
    _comb2_body, out_shape=jax.ShapeDtypeStruct((N, D), jnp.float32))


def kernel(V, E, X, W, b):
    del V
    src = E[0].astype(jnp.int32)
    dst = E[1].astype(jnp.int32)
    npad = EDGES_P - EDGES
    ppt = npad // NW
    pad_ar = jnp.arange(npad, dtype=jnp.int32)
    src_p = jnp.concatenate(
        [src.reshape(NW, EPT), (N + (pad_ar % 8)).reshape(NW, ppt)],
        axis=1).reshape(-1)
    dst_p = jnp.concatenate(
        [dst.reshape(NW, EPT), (pad_ar % N).reshape(NW, ppt)],
        axis=1).reshape(-1)
    dstf = dst.reshape(NW, EPT)
    zeros = jnp.zeros((N, D), jnp.float32)

    dp = _deg_kernel(dstf)
    y0 = _prep(X, W, dp)
    p = _hop_kernel(y0, zeros, src_p, dst_p)
    y1 = _comb1(p, dp)
    q = _hop_kernel(y1, zeros, src_p, dst_p)
    return _comb2(q, dp, b.reshape(1, D))


# speedup vs baseline: 1.0439x; 1.0439x over previous
import functools

import jax
import jax.numpy as jnp
from jax import lax
from jax.experimental import pallas as pl
from jax.experimental.pallas import tpu as pltpu
from jax.experimental.pallas import tpu_sc as plsc

N = 10000
EDGES = 320000
D = 128
NC, NS = 2, 16
NW = NC * NS
EPT = EDGES // NW
CHUNK = 64
NCHUNK = 160
EPT_P = NCHUNK * CHUNK
EDGES_P = NW * EPT_P
RING = 4
PDIST = 3
NP = N + 8
ROWS_PT = 624
TAIL_ROWS = N - NS * ROWS_PT

_MESH = plsc.VectorSubcoreMesh(
    core_axis_name="c", subcore_axis_name="s", num_cores=NC, num_subcores=NS)


@functools.partial(
    pl.kernel,
    out_type=jax.ShapeDtypeStruct((NW, N), jnp.float32),
    mesh=_MESH,
    scratch_types=[
        pltpu.VMEM((EPT,), jnp.int32),
        pltpu.VMEM((N,), jnp.float32),
    ],
    compiler_params=pltpu.CompilerParams(needs_layout_passes=False),
)
def _deg_kernel(dst_hbm, out_hbm, dbuf, acc):
    cid = lax.axis_index("c")
    sid = lax.axis_index("s")
    wid = cid * NS + sid
    pltpu.sync_copy(dst_hbm.at[wid], dbuf)
    zeros16 = jnp.zeros((16,), jnp.float32)

    @pl.loop(0, N // 16)
    def _zero(i):
        acc[pl.ds(i * 16, 16)] = zeros16

    ones16 = jnp.ones((16,), jnp.float32)

    @pl.loop(0, EPT // 16)
    def _hist(g):
        idx = dbuf[pl.ds(g * 16, 16)]
        plsc.addupdate_scatter(acc, [idx], ones16)

    pltpu.sync_copy(acc, out_hbm.at[wid])


@functools.partial(
    pl.kernel,
    out_type=jax.ShapeDtypeStruct((NC, N, D), jnp.float32),
    mesh=_MESH,
    scratch_types=[
        pltpu.VMEM_SHARED((N, D), jnp.float32),
        pltpu.VMEM((EPT_P,), jnp.int32),
        pltpu.VMEM((RING, CHUNK), jnp.int32),
        pltpu.VMEM((RING, CHUNK, D), jnp.float32),
        pltpu.SemaphoreType.DMA((RING,)),
        pltpu.SemaphoreType.DMA((RING,)),
        pltpu.SemaphoreType.DMA((RING,)),
    ],
)
def _hop_kernel(y_hbm, zero_hbm, src_hbm, dst_hbm, out_hbm,
                acc, sidx, didx, rows, gsem, isem, ssem):
    cid = lax.axis_index("c")
    sid = lax.axis_index("s")
    wid = cid * NS + sid
    row0 = pl.multiple_of(sid * ROWS_PT, 8)
    ebase = pl.multiple_of(wid * EPT_P, 64)

    def start_chunk(jj, b):
        eoff = pl.multiple_of(ebase + jj * CHUNK, 64)
        pltpu.async_copy(dst_hbm.at[pl.ds(eoff, CHUNK)], didx.at[b],
                         isem.at[b])
        loff = pl.multiple_of(jj * CHUNK, 64)
        pltpu.async_copy(y_hbm.at[sidx.at[pl.ds(loff, CHUNK)]], rows.at[b],
                         gsem.at[b])

    def wait_chunk(jj, b):
        eoff = pl.multiple_of(ebase + jj * CHUNK, 64)
        pltpu.make_async_copy(dst_hbm.at[pl.ds(eoff, CHUNK)], didx.at[b],
                              isem.at[b]).wait()
        loff = pl.multiple_of(jj * CHUNK, 64)
        pltpu.make_async_copy(y_hbm.at[sidx.at[pl.ds(loff, CHUNK)]],
                              rows.at[b], gsem.at[b]).wait()

    pltpu.sync_copy(src_hbm.at[pl.ds(ebase, EPT_P)], sidx)
    for b in range(PDIST):
        start_chunk(b, b)

    @pl.when(cid == 0)
    def _():
        pltpu.sync_copy(y_hbm.at[pl.ds(row0, ROWS_PT)],
                        acc.at[pl.ds(row0, ROWS_PT)])

        @pl.when(sid == NS - 1)
        def _():
            pltpu.sync_copy(y_hbm.at[pl.ds(NS * ROWS_PT, TAIL_ROWS)],
                            acc.at[pl.ds(NS * ROWS_PT, TAIL_ROWS)])

    @pl.when(cid != 0)
    def _():
        pltpu.sync_copy(zero_hbm.at[pl.ds(row0, ROWS_PT)],
                        acc.at[pl.ds(row0, ROWS_PT)])

        @pl.when(sid == NS - 1)
        def _():
            pltpu.sync_copy(zero_hbm.at[pl.ds(NS * ROWS_PT, TAIL_ROWS)],
                            acc.at[pl.ds(NS * ROWS_PT, TAIL_ROWS)])

    plsc.subcore_barrier()

    @pl.loop(0, NCHUNK, step=RING)
    def _main(j):
        for b in range(RING):
            k = j + b
            q = (b + PDIST) % RING
            kp = k + PDIST

            @pl.when(kp < NCHUNK)
            def _():
                @pl.when(k >= RING - PDIST)
                def _():
                    pltpu.make_async_copy(
                        rows.at[q], acc.at[didx.at[q]], ssem.at[q]).wait()

                start_chunk(kp, q)

            wait_chunk(k, b)
            pltpu.async_copy(rows.at[b], acc.at[didx.at[b]], ssem.at[b],
                             add=True)

    for b in range(RING):
        pltpu.make_async_copy(
            rows.at[b], acc.at[didx.at[b]], ssem.at[b]).wait()

    plsc.subcore_barrier()
    pltpu.sync_copy(acc.at[pl.ds(row0, ROWS_PT)],
                    out_hbm.at[cid, pl.ds(row0, ROWS_PT)])

    @pl.when(sid == NS - 1)
    def _():
        pltpu.sync_copy(acc.at[pl.ds(NS * ROWS_PT, TAIL_ROWS)],
                        out_hbm.at[cid, pl.ds(NS * ROWS_PT, TAIL_ROWS)])


def _deg_from(dp):
    return jnp.sum(dp, axis=0) + 1.0


def _prep_body(x_ref, w_ref, dp_ref, y0_ref):
    dinv = lax.rsqrt(_deg_from(dp_ref[...]))
    xw = lax.dot_general(x_ref[...], w_ref[...], (((1,), (1,)), ((), ())),
                         preferred_element_type=jnp.float32)
    y0_ref[pl.ds(0, N)] = dinv[:, None] * xw
    y0_ref[pl.ds(N, NP - N)] = jnp.zeros((NP - N, D), jnp.float32)


def _comb1_body(p_ref, dp_ref, y1_ref):
    deg = _deg_from(dp_ref[...])
    y1_ref[pl.ds(0, N)] = (p_ref[0] + p_ref[1]) / deg[:, None]
    y1_ref[pl.ds(N, NP - N)] = jnp.zeros((NP - N, D), jnp.float32)


def _comb2_body(p_ref, dp_ref, b_ref, o_ref):
    dinv = lax.rsqrt(_deg_from(dp_ref[...]))
    o_ref[...] = dinv[:, None] * (p_ref[0] + p_ref[1]) + b_ref[...]


_prep = pl.pallas_call(
    _prep_body, out_shape=jax.ShapeDtypeStruct((NP, D), jnp.float32))
_comb1 = pl.pallas_call(
    _comb1_body, out_shape=jax.ShapeDtypeStruct((NP, D), jnp.float32))
_comb2 = pl.pallas_call(
    _comb2_body, out_shape=jax.ShapeDtypeStruct((N, D), jnp.float32))


def kernel(V, E, X, W, b):
    del V
    src = E[0].astype(jnp.int32)
    dst = E[1].astype(jnp.int32)
    npad = EDGES_P - EDGES
    ppt = npad // NW
    pad_ar = jnp.arange(npad, dtype=jnp.int32)
    src_p = jnp.concatenate(
        [src.reshape(NW, EPT), (N + (pad_ar % 8)).reshape(NW, ppt)],
        axis=1).reshape(-1)
    dst_p = jnp.concatenate(
        [dst.reshape(NW, EPT), (pad_ar % N).reshape(NW, ppt)],
        axis=1).reshape(-1)
    dstf = dst.reshape(NW, EPT)
    zeros = jnp.zeros((N, D), jnp.float32)

    dp = _deg_kernel(dstf)
    y0 = _prep(X, W, dp)
    p = _hop_kernel(y0, zeros, src_p, dst_p)
    y1 = _comb1(p, dp)
    q = _hop_kernel(y1, zeros, src_p, dst_p)
    return _comb2(q, dp, b.reshape(1, D))
